# Initial kernel scaffold; baseline (speedup 1.0000x reference)
import jax
import jax.numpy as jnp
from jax.experimental import pallas as pl


def kernel(h, pos, edge_emb_W, edge_emb_b, node_W1, node_b1, node_W2, node_b2, edgeN_W1, edgeN_b1, edgeN_W2, edgeN_b2, msg_W, msg_b, cent_W, cent_b, ln_g, ln_b, out_W, out_b, edge_index, is_mol):
    raise NotImplementedError("write your pallas kernel here")


# speedup vs baseline: 6.1168x; 6.1168x over previous
import functools

import jax
import jax.numpy as jnp
from jax import lax
from jax.experimental import pallas as pl
from jax.experimental.pallas import tpu as pltpu
from jax.experimental.pallas import tpu_sc as plsc

N = 10000
E = 320000
NODE_DIM = 128
EDGE_DIM = 64
HID = 128
NB = 4
NG = 20
CUTOFF = 10.0

NC = 2
NS = 16
NW = NC * NS
CH = 128
T = E // CH
CPW = (T + NW - 1) // NW
NP = 10240

_MESH = plsc.VectorSubcoreMesh(
    core_axis_name="c", subcore_axis_name="s", num_cores=NC, num_subcores=NS)

ROWS_PER_SUB = NP // NS


@functools.partial(
    pl.kernel,
    out_type=[
        jax.ShapeDtypeStruct((E, 8), jnp.float32),
        jax.ShapeDtypeStruct((NW, 1, NP), jnp.float32),
    ],
    mesh=_MESH,
    compiler_params=pltpu.CompilerParams(needs_layout_passes=False),
    scratch_types=[
        pltpu.VMEM((N * 4,), jnp.float32),
        pltpu.VMEM((CH,), jnp.int32),
        pltpu.VMEM((CH,), jnp.int32),
        pltpu.VMEM((CH, 8), jnp.float32),
        pltpu.VMEM((1, NP), jnp.float32),
        pltpu.SemaphoreType.DMA,
    ],
)
def _geo_sc(posf, rowi, coli, geo, degp,
            pos_v, idx_r, idx_c, buf, deg_v, sem):
    c = lax.axis_index("c")
    s = lax.axis_index("s")
    wid = s * NC + c

    pltpu.sync_copy(posf, pos_v)

    zf = jnp.zeros((16,), jnp.float32)

    def zdeg(j, carry):
        deg_v[0, pl.ds(j * 16, 16)] = zf
        return carry
    lax.fori_loop(0, NP // 16, zdeg, 0)

    lane = lax.broadcasted_iota(jnp.int32, (16,), 0)
    zi = jnp.zeros((16,), jnp.int32)
    onesf = jnp.full((16,), 1.0, jnp.float32)

    def chunk(i, carry):
        t = wid + i * NW

        @pl.when(t < T)
        def _():
            base = t * CH
            pltpu.sync_copy(rowi.at[pl.ds(base, CH)], idx_r)
            pltpu.sync_copy(coli.at[pl.ds(base, CH)], idx_c)
            for g in range(CH // 16):
                sl = pl.ds(g * 16, 16)
                ir4 = idx_r[sl] * 4
                ic4 = idx_c[sl] * 4
                dx = plsc.load_gather(pos_v, [ir4]) - plsc.load_gather(pos_v, [ic4])
                dy = (plsc.load_gather(pos_v, [ir4 + 1])
                      - plsc.load_gather(pos_v, [ic4 + 1]))
                dz = (plsc.load_gather(pos_v, [ir4 + 2])
                      - plsc.load_gather(pos_v, [ic4 + 2]))
                d2 = dx * dx + dy * dy + dz * dz
                mr = plsc.load_gather(pos_v, [ir4 + 3])
                mc = plsc.load_gather(pos_v, [ic4 + 3])
                eids = g * 16 + lane
                plsc.store_scatter(buf, [eids, zi], d2)
                plsc.store_scatter(buf, [eids, zi + 1], mr)
                plsc.store_scatter(buf, [eids, zi + 2], mc)
                plsc.addupdate_scatter(deg_v, [zi, idx_r[sl]], onesf)
            pltpu.sync_copy(buf, geo.at[pl.ds(base, CH)])
        return carry
    lax.fori_loop(0, CPW, chunk, 0)

    pltpu.sync_copy(deg_v, degp.at[wid])


@functools.partial(
    pl.kernel,
    out_type=jax.ShapeDtypeStruct((NC, NP, HID), jnp.float32),
    mesh=_MESH,
    compiler_params=pltpu.CompilerParams(needs_layout_passes=False),
    scratch_types=[
        pltpu.VMEM((CH,), jnp.int32),
        pltpu.VMEM((CH,), jnp.int32),
        pltpu.VMEM((CH, HID), jnp.float32),
        pltpu.VMEM((CH, HID), jnp.float32),
        pltpu.VMEM_SHARED((NP, HID), jnp.float32),
        pltpu.SemaphoreType.DMA,
        pltpu.SemaphoreType.DMA,
    ],
)
def _edge_sc(hn, he, rowi, coli, zeros128, partials,
             idx_r, idx_c, hn_v, he_v, aggr_sh, gsem, esem):
    c = lax.axis_index("c")
    s = lax.axis_index("s")
    wid = s * NC + c

    @pl.when(s == 0)
    def _():
        pltpu.sync_copy(zeros128, aggr_sh)
    plsc.subcore_barrier()

    def chunk(i, carry):
        t = wid + i * NW

        @pl.when(t < T)
        def _():
            base = t * CH
            pltpu.sync_copy(coli.at[pl.ds(base, CH)], idx_c)
            gcp = pltpu.async_copy(hn.at[idx_c], hn_v, gsem)
            ecp = pltpu.async_copy(he.at[pl.ds(base, CH)], he_v, esem)
            pltpu.sync_copy(rowi.at[pl.ds(base, CH)], idx_r)
            gcp.wait()
            ecp.wait()

            def mulrow(e, carry2):
                for k in range(HID // 16):
                    sl = pl.ds(k * 16, 16)
                    he_v[e, sl] = he_v[e, sl] * hn_v[e, sl]
                return carry2
            lax.fori_loop(0, CH, mulrow, 0)
            pltpu.sync_copy(he_v, aggr_sh.at[idx_r], add=True)
        return carry
    lax.fori_loop(0, CPW, chunk, 0)

    plsc.subcore_barrier()
    pltpu.sync_copy(aggr_sh.at[pl.ds(s * ROWS_PER_SUB, ROWS_PER_SUB)],
                    partials.at[c, pl.ds(s * ROWS_PER_SUB, ROWS_PER_SUB)])


BE = 2000
BN = 2048


def _edge_mlp_body(g_ref, eW, eb, W1, b1, W2, b2, out_ref):
    g = g_ref[...]
    d2 = g[:, 0:1]
    el = jnp.sqrt(d2 + 1e-12)
    step = CUTOFF / (NG - 1)
    offs = (jnp.arange(NG, dtype=jnp.int32).astype(jnp.float32) * step)[None, :]
    coeff = -0.5 / step**2
    smear = jnp.exp(coeff * (el - offs) ** 2)
    attr = jnp.concatenate([smear, g[:, 1:2], g[:, 2:3]], axis=1)
    ea = jnp.dot(attr, eW[...], preferred_element_type=jnp.float32) + eb[...]
    m = jnp.maximum(
        jnp.dot(ea, W1[...], preferred_element_type=jnp.float32) + b1[...], 0.0)
    out_ref[...] = (
        jnp.dot(m, W2[...], preferred_element_type=jnp.float32) + b2[...])


def _edge_mlp(geo, eW, eb, W1, b1, W2, b2):
    grid = (E // BE,)
    return pl.pallas_call(
        _edge_mlp_body,
        grid=grid,
        in_specs=[
            pl.BlockSpec((BE, 8), lambda i: (i, 0)),
            pl.BlockSpec(eW.shape, lambda i: (0, 0)),
            pl.BlockSpec(eb.shape, lambda i: (0, 0)),
            pl.BlockSpec(W1.shape, lambda i: (0, 0)),
            pl.BlockSpec(b1.shape, lambda i: (0, 0)),
            pl.BlockSpec(W2.shape, lambda i: (0, 0)),
            pl.BlockSpec(b2.shape, lambda i: (0, 0)),
        ],
        out_specs=pl.BlockSpec((BE, HID), lambda i: (i, 0)),
        out_shape=jax.ShapeDtypeStruct((E, HID), jnp.float32),
    )(geo, eW, eb, W1, b1, W2, b2)


def _node_mlp_body(h_ref, W1, b1, W2, b2, out_ref):
    m = jnp.maximum(
        jnp.dot(h_ref[...], W1[...], preferred_element_type=jnp.float32)
        + b1[...], 0.0)
    out_ref[...] = (
        jnp.dot(m, W2[...], preferred_element_type=jnp.float32) + b2[...])


def _node_mlp(h, W1, b1, W2, b2):
    grid = ((N + BN - 1) // BN,)
    return pl.pallas_call(
        _node_mlp_body,
        grid=grid,
        in_specs=[
            pl.BlockSpec((BN, NODE_DIM), lambda i: (i, 0)),
            pl.BlockSpec(W1.shape, lambda i: (0, 0)),
            pl.BlockSpec(b1.shape, lambda i: (0, 0)),
            pl.BlockSpec(W2.shape, lambda i: (0, 0)),
            pl.BlockSpec(b2.shape, lambda i: (0, 0)),
        ],
        out_specs=pl.BlockSpec((BN, HID), lambda i: (i, 0)),
        out_shape=jax.ShapeDtypeStruct((N, HID), jnp.float32),
    )(h, W1, b1, W2, b2)


def _combine_body(h_ref, p0, p1, dg, msgW, msgb, centW, centb,
                  lng, lnb, outW, outb, out_ref):
    h = h_ref[...]
    aggr = p0[...] + p1[...]
    deg = jnp.sum(dg[...], axis=(0, 1))[:, None]
    am = (jnp.dot(aggr, msgW[...], preferred_element_type=jnp.float32)
          + deg * msgb[...])
    o = (jnp.dot(h, centW[...], preferred_element_type=jnp.float32)
         + centb[...] + am)
    mu = jnp.mean(o, axis=-1, keepdims=True)
    var = jnp.mean((o - mu) ** 2, axis=-1, keepdims=True)
    o = (o - mu) / jnp.sqrt(var + 1e-5) * lng[...] + lnb[...]
    o = jnp.maximum(o, 0.0)
    out_ref[...] = (
        h + jnp.dot(o, outW[...], preferred_element_type=jnp.float32)
        + outb[...])


def _combine(h, p0, p1, dg, msgW, msgb, centW, centb, lng, lnb,
             outW, outb):
    grid = ((N + BN - 1) // BN,)
    full = lambda a: pl.BlockSpec(a.shape, lambda i: (0,) * a.ndim)
    return pl.pallas_call(
        _combine_body,
        grid=grid,
        in_specs=[
            pl.BlockSpec((BN, NODE_DIM), lambda i: (i, 0)),
            pl.BlockSpec((BN, HID), lambda i: (i, 0)),
            pl.BlockSpec((BN, HID), lambda i: (i, 0)),
            pl.BlockSpec((NW, 1, BN), lambda i: (0, 0, i)),
            full(msgW), full(msgb), full(centW), full(centb),
            full(lng), full(lnb), full(outW), full(outb),
        ],
        out_specs=pl.BlockSpec((BN, NODE_DIM), lambda i: (i, 0)),
        out_shape=jax.ShapeDtypeStruct((N, NODE_DIM), jnp.float32),
    )(h, p0, p1, dg, msgW, msgb, centW, centb, lng, lnb, outW, outb)


def kernel(h, pos, edge_emb_W, edge_emb_b, node_W1, node_b1, node_W2,
           node_b2, edgeN_W1, edgeN_b1, edgeN_W2, edgeN_b2, msg_W, msg_b,
           cent_W, cent_b, ln_g, ln_b, out_W, out_b, edge_index, is_mol):
    rowi = edge_index[0]
    coli = edge_index[1]
    posf = jnp.concatenate(
        [pos, is_mol.astype(jnp.float32)[:, None]], axis=1).reshape(-1)
    zeros128 = jnp.zeros((NP, HID), jnp.float32)

    geo, degp = _geo_sc(posf, rowi, coli)

    eb2 = edge_emb_b[None, :]
    hcur = h
    for i in range(NB):
        hn = _node_mlp(hcur, node_W1[i], node_b1[i][None, :],
                       node_W2[i], node_b2[i][None, :])
        he = _edge_mlp(geo, edge_emb_W, eb2,
                       edgeN_W1[i], edgeN_b1[i][None, :],
                       edgeN_W2[i], edgeN_b2[i][None, :])
        parts = _edge_sc(hn, he, rowi, coli, zeros128)
        hcur = _combine(hcur, parts[0], parts[1], degp,
                        msg_W[i], msg_b[i][None, :],
                        cent_W[i], cent_b[i][None, :],
                        ln_g[i][None, :], ln_b[i][None, :],
                        out_W[i], out_b[i][None, :])
    return hcur
